# baseline (speedup 1.0000x reference)
import jax
import jax.numpy as jnp
import numpy as np
from jax import lax
from jax.experimental import pallas as pl
from jax.experimental.pallas import tpu as pltpu
from jax.experimental.pallas import tpu_sc as plsc

_L = 5
_K = 5
_W = 30.0
_MULT = [(1000003 ** k) % (2 ** 31 - 1) for k in range(_K)]
_C = float(np.log2(np.e) / _W)

_TN = 4000
_NW = 32
_BPW = 25
_RPW = _BPW * 128
_NPAD = _NW * _RPW


def _sc_hash_body(reft_hbm, a_hbm, b_hbm, out_hbm,
                  a_v, b_v, rblk, stage, sem):
    wid = lax.axis_index("s") * 2 + lax.axis_index("c")
    base = wid * _RPW

    pltpu.sync_copy(a_hbm, a_v)
    pltpu.sync_copy(b_hbm, b_v)

    def slab(bi, carry):
        pltpu.sync_copy(reft_hbm.at[:, pl.ds(base + bi * 128, 128)], rblk)

        def group(g, carry2):
            accs = [b_v[pl.ds(j * 16, 16)] for j in range(_L * _K)]
            for d in range(64):
                xd = rblk[d, pl.ds(g * 16, 16)]
                for j in range(_L * _K):
                    accs[j] = accs[j] + xd * a_v[pl.ds((d * _L * _K + j) * 16,
                                                       16)]
            one = jnp.full((16,), 1, jnp.int32)
            zero = jnp.full((16,), 0, jnp.int32)
            for l in range(_L):
                idl = zero
                for k in range(_K):
                    j = _K * l + k
                    t = accs[j] / _W
                    it = t.astype(jnp.int32)
                    tf = it.astype(jnp.float32)
                    it = it - jnp.where(tf > t, one, zero)
                    idl = idl + it * np.int32(_MULT[k])
                stage[pl.ds(l * _RPW + bi * 128 + g * 16, 16)] = idl
            return carry2

        lax.fori_loop(0, 8, group, 0)
        return carry

    lax.fori_loop(0, _BPW, slab, 0)
    for l in range(_L):
        pltpu.sync_copy(stage.at[pl.ds(l * _RPW, _RPW)],
                        out_hbm.at[pl.ds(l * _NPAD + base, _RPW)])


def _sc_hash(refT, A, b):
    mesh = plsc.VectorSubcoreMesh(core_axis_name="c", subcore_axis_name="s")
    k = pl.kernel(
        _sc_hash_body,
        mesh=mesh,
        out_type=jax.ShapeDtypeStruct((8 * _NPAD,), jnp.int32),
        scratch_types=[
            pltpu.VMEM((64 * _L * _K * 16,), jnp.float32),
            pltpu.VMEM((_L * _K * 16,), jnp.float32),
            pltpu.VMEM((64, 128), jnp.float32),
            pltpu.VMEM((_L * _RPW,), jnp.int32),
            pltpu.SemaphoreType.DMA,
        ],
    )
    a_bc = jnp.tile(A.reshape(-1, 1), (1, 16)).reshape(-1)
    b_bc = jnp.tile(b.reshape(-1, 1), (1, 16)).reshape(-1)
    return k(refT, a_bc, b_bc)


def _prep_body(u_ref, a_ref, b_ref, qb_ref):
    U = u_ref[...]
    hq = jnp.floor((jax.lax.dot_general(
        U, a_ref[...], (((1,), (0,)), ((), ())),
        preferred_element_type=jnp.float32) + b_ref[...].reshape(1, -1))
        / _W).astype(jnp.int32)
    for l in range(_L):
        acc = hq[:, _K * l:_K * l + 1] * np.int32(_MULT[0])
        for k in range(1, _K):
            acc = acc + hq[:, _K * l + k:_K * l + k + 1] * np.int32(_MULT[k])
        qb_ref[:, l:l + 1] = acc
    qb_ref[:, _L:] = jnp.zeros_like(qb_ref[:, _L:])


def _fused_body(u_ref, uc_ref, qb_ref, rbt_ref, ref_ref,
                out_ref, num_ref, den_ref):
    i = pl.program_id(0)
    nt = pl.num_programs(0)

    R = ref_ref[...]
    rbt = rbt_ref[0]

    match = qb_ref[:, 0:1] == rbt[0:1, :]
    for l in range(1, _L):
        match = match | (qb_ref[:, l:l + 1] == rbt[l:l + 1, :])

    rrow = jax.lax.dot_general(
        jnp.full((1, R.shape[1]), -0.5 * _C, jnp.float32), R * R,
        (((1,), (1,)), ((), ())),
        preferred_element_type=jnp.float32)

    Rb = R.astype(jnp.bfloat16)
    S = jax.lax.dot_general(uc_ref[...], Rb, (((1,), (1,)), ((), ())),
                            preferred_element_type=jnp.float32)
    w = jnp.where(match, jnp.exp2(S + rrow), 0.0).astype(jnp.bfloat16)

    pnum = jax.lax.dot_general(w, Rb, (((1,), (0,)), ((), ())),
                               preferred_element_type=jnp.float32)
    pden = jax.lax.dot_general(
        w, jnp.ones((_TN, 1), jnp.bfloat16), (((1,), (0,)), ((), ())),
        preferred_element_type=jnp.float32)

    @pl.when(i == 0)
    def _init():
        num_ref[...] = pnum
        den_ref[...] = pden

    @pl.when(i > 0)
    def _acc():
        num_ref[...] += pnum
        den_ref[...] += pden

    @pl.when(i == nt - 1)
    def _final():
        U = u_ref[...]
        un2 = jnp.sum(U * U, axis=1, keepdims=True)
        eps = jnp.exp2(un2 * (0.5 * _C)) * 1e-6
        out_ref[...] = num_ref[...] / (den_ref[...] + eps) - U


@jax.jit
def kernel(U, ref, A, b):
    Q, D = U.shape
    N = ref.shape[0]
    assert N % _TN == 0
    grid = (N // _TN,)
    Uc = (U * jnp.float32(_C)).astype(jnp.bfloat16)
    bcol = b.reshape(-1, 1)

    refTp = jnp.pad(ref.T, ((0, 0), (0, _NPAD - N)))
    rbt = _sc_hash(refTp, A, b)
    rbt3 = rbt.reshape(8, _NPAD)[:, :N].reshape(
        8, N // _TN, _TN).transpose(1, 0, 2)

    qb = pl.pallas_call(
        _prep_body,
        in_specs=[
            pl.BlockSpec((Q, D), lambda: (0, 0)),
            pl.BlockSpec((D, _L * _K), lambda: (0, 0)),
            pl.BlockSpec((_L * _K, 1), lambda: (0, 0)),
        ],
        out_shape=jax.ShapeDtypeStruct((Q, 8), jnp.int32),
    )(U, A, bcol)

    out = pl.pallas_call(
        _fused_body,
        grid=grid,
        in_specs=[
            pl.BlockSpec((Q, D), lambda i: (0, 0)),
            pl.BlockSpec((Q, D), lambda i: (0, 0)),
            pl.BlockSpec((Q, 8), lambda i: (0, 0)),
            pl.BlockSpec((1, 8, _TN), lambda i: (i, 0, 0)),
            pl.BlockSpec((_TN, D), lambda i: (i, 0)),
        ],
        out_shape=jax.ShapeDtypeStruct((Q, D), jnp.float32),
        scratch_shapes=[
            pltpu.VMEM((Q, D), jnp.float32),
            pltpu.VMEM((Q, 1), jnp.float32),
        ],
    )(U, Uc, qb, rbt3, ref)
    return out


# speedup vs baseline: 4.3300x; 4.3300x over previous
import jax
import jax.numpy as jnp
import numpy as np
from jax.experimental import pallas as pl
from jax.experimental.pallas import tpu as pltpu

_L = 5
_K = 5
_W = 30.0
_MULT = np.array([(1000003 ** k) % (2 ** 31 - 1) for k in range(_K)],
                 dtype=np.int32)
_MULT_COL = np.tile(_MULT, _L).reshape(_L * _K, 1)
_C = float(np.log2(np.e) / _W)

_TN = 2000


def _prep_body(u_ref, a_ref, b_ref, qb_ref):
    U = u_ref[...]
    hq = jnp.floor((jax.lax.dot_general(
        U, a_ref[...], (((1,), (0,)), ((), ())),
        preferred_element_type=jnp.float32) + b_ref[...].reshape(1, -1))
        / _W).astype(jnp.int32)
    for l in range(_L):
        acc = hq[:, _K * l:_K * l + 1] * _MULT[0]
        for k in range(1, _K):
            acc = acc + hq[:, _K * l + k:_K * l + k + 1] * _MULT[k]
        qb_ref[:, l:l + 1] = acc
    qb_ref[:, _L:] = jnp.zeros_like(qb_ref[:, _L:])


def _fused_body(u_ref, uc_ref, qb_ref, ref_ref, a_ref, b_ref,
                mult_ref, out_ref, num_ref, den_ref):
    i = pl.program_id(0)
    nt = pl.num_programs(0)

    R = ref_ref[...]
    A = a_ref[...]
    bcol = b_ref[...]

    hr = jnp.floor((jax.lax.dot_general(
        A, R, (((0,), (1,)), ((), ())),
        preferred_element_type=jnp.float32) + bcol) / _W).astype(jnp.int32)
    hm = hr * mult_ref[...]
    rb = []
    for l in range(_L):
        acc = hm[_K * l:_K * l + 1, :]
        for k in range(1, _K):
            acc = acc + hm[_K * l + k:_K * l + k + 1, :]
        rb.append(acc)

    match = qb_ref[:, 0:1] == rb[0]
    for l in range(1, _L):
        match = match | (qb_ref[:, l:l + 1] == rb[l])

    rrow = jax.lax.dot_general(
        jnp.full((1, R.shape[1]), -0.5 * _C, jnp.float32), R * R,
        (((1,), (1,)), ((), ())),
        preferred_element_type=jnp.float32)

    Rb = R.astype(jnp.bfloat16)
    S = jax.lax.dot_general(uc_ref[...], Rb, (((1,), (1,)), ((), ())),
                            preferred_element_type=jnp.float32)
    w = jnp.where(match, jnp.exp2(S + rrow), 0.0).astype(jnp.bfloat16)

    pnum = jax.lax.dot_general(w, Rb, (((1,), (0,)), ((), ())),
                               preferred_element_type=jnp.float32)
    pden = jax.lax.dot_general(
        w, jnp.ones((_TN, 1), jnp.bfloat16), (((1,), (0,)), ((), ())),
        preferred_element_type=jnp.float32)

    @pl.when(i == 0)
    def _init():
        num_ref[...] = pnum
        den_ref[...] = pden

    @pl.when(i > 0)
    def _acc():
        num_ref[...] += pnum
        den_ref[...] += pden

    @pl.when(i == nt - 1)
    def _final():
        U = u_ref[...]
        un2 = jnp.sum(U * U, axis=1, keepdims=True)
        eps = jnp.exp2(un2 * (0.5 * _C)) * 1e-6
        out_ref[...] = num_ref[...] / (den_ref[...] + eps) - U


@jax.jit
def kernel(U, ref, A, b):
    Q, D = U.shape
    N = ref.shape[0]
    assert N % _TN == 0
    grid = (N // _TN,)
    Uc = (U * jnp.float32(_C)).astype(jnp.bfloat16)
    bcol = b.reshape(-1, 1)
    mult = jnp.asarray(_MULT_COL)

    qb = pl.pallas_call(
        _prep_body,
        in_specs=[
            pl.BlockSpec((Q, D), lambda: (0, 0)),
            pl.BlockSpec((D, _L * _K), lambda: (0, 0)),
            pl.BlockSpec((_L * _K, 1), lambda: (0, 0)),
        ],
        out_shape=jax.ShapeDtypeStruct((Q, 8), jnp.int32),
    )(U, A, bcol)

    out = pl.pallas_call(
        _fused_body,
        grid=grid,
        in_specs=[
            pl.BlockSpec((Q, D), lambda i: (0, 0)),
            pl.BlockSpec((Q, D), lambda i: (0, 0)),
            pl.BlockSpec((Q, 8), lambda i: (0, 0)),
            pl.BlockSpec((_TN, D), lambda i: (i, 0)),
            pl.BlockSpec((D, _L * _K), lambda i: (0, 0)),
            pl.BlockSpec((_L * _K, 1), lambda i: (0, 0)),
            pl.BlockSpec((_L * _K, 1), lambda i: (0, 0)),
        ],
        out_shape=jax.ShapeDtypeStruct((Q, D), jnp.float32),
        scratch_shapes=[
            pltpu.VMEM((Q, D), jnp.float32),
            pltpu.VMEM((Q, 1), jnp.float32),
        ],
    )(U, Uc, qb, ref, A, bcol, mult)
    return out


# speedup vs baseline: 4.5141x; 1.0425x over previous
import jax
import jax.numpy as jnp
import numpy as np
from jax.experimental import pallas as pl
from jax.experimental.pallas import tpu as pltpu

_L = 5
_K = 5
_W = 30.0
_MULT = np.array([(1000003 ** k) % (2 ** 31 - 1) for k in range(_K)],
                 dtype=np.int32)
_MULT_COL = np.tile(_MULT, _L).reshape(_L * _K, 1)
_C = float(np.log2(np.e) / _W)

_TN = 4000


def _prep_body(u_ref, a_ref, b_ref, qb_ref):
    U = u_ref[...]
    hq = jnp.floor((jax.lax.dot_general(
        U, a_ref[...], (((1,), (0,)), ((), ())),
        preferred_element_type=jnp.float32) + b_ref[...].reshape(1, -1))
        / _W).astype(jnp.int32)
    for l in range(_L):
        acc = hq[:, _K * l:_K * l + 1] * _MULT[0]
        for k in range(1, _K):
            acc = acc + hq[:, _K * l + k:_K * l + k + 1] * _MULT[k]
        qb_ref[:, l:l + 1] = acc
    qb_ref[:, _L:] = jnp.zeros_like(qb_ref[:, _L:])


def _fused_body(u_ref, uc_ref, qb_ref, ref_ref, a_ref, b_ref,
                mult_ref, out_ref, num_ref, den_ref):
    i = pl.program_id(0)
    nt = pl.num_programs(0)

    R = ref_ref[...]
    A = a_ref[...]
    bcol = b_ref[...]

    hr = jnp.floor((jax.lax.dot_general(
        A, R, (((0,), (1,)), ((), ())),
        preferred_element_type=jnp.float32) + bcol) / _W).astype(jnp.int32)
    hm = hr * mult_ref[...]
    rb = []
    for l in range(_L):
        acc = hm[_K * l:_K * l + 1, :]
        for k in range(1, _K):
            acc = acc + hm[_K * l + k:_K * l + k + 1, :]
        rb.append(acc)

    match = qb_ref[:, 0:1] == rb[0]
    for l in range(1, _L):
        match = match | (qb_ref[:, l:l + 1] == rb[l])

    rrow = jax.lax.dot_general(
        jnp.full((1, R.shape[1]), -0.5 * _C, jnp.float32), R * R,
        (((1,), (1,)), ((), ())),
        preferred_element_type=jnp.float32)

    Rb = R.astype(jnp.bfloat16)
    S = jax.lax.dot_general(uc_ref[...], Rb, (((1,), (1,)), ((), ())),
                            preferred_element_type=jnp.float32)
    w = jnp.where(match, jnp.exp2(S + rrow), 0.0).astype(jnp.bfloat16)

    pnum = jax.lax.dot_general(w, Rb, (((1,), (0,)), ((), ())),
                               preferred_element_type=jnp.float32)
    pden = jax.lax.dot_general(
        w, jnp.ones((_TN, 1), jnp.bfloat16), (((1,), (0,)), ((), ())),
        preferred_element_type=jnp.float32)

    @pl.when(i == 0)
    def _init():
        num_ref[...] = pnum
        den_ref[...] = pden

    @pl.when(i > 0)
    def _acc():
        num_ref[...] += pnum
        den_ref[...] += pden

    @pl.when(i == nt - 1)
    def _final():
        U = u_ref[...]
        un2 = jnp.sum(U * U, axis=1, keepdims=True)
        eps = jnp.exp2(un2 * (0.5 * _C)) * 1e-6
        out_ref[...] = num_ref[...] / (den_ref[...] + eps) - U


@jax.jit
def kernel(U, ref, A, b):
    Q, D = U.shape
    N = ref.shape[0]
    assert N % _TN == 0
    grid = (N // _TN,)
    Uc = (U * jnp.float32(_C)).astype(jnp.bfloat16)
    bcol = b.reshape(-1, 1)
    mult = jnp.asarray(_MULT_COL)

    qb = pl.pallas_call(
        _prep_body,
        in_specs=[
            pl.BlockSpec((Q, D), lambda: (0, 0)),
            pl.BlockSpec((D, _L * _K), lambda: (0, 0)),
            pl.BlockSpec((_L * _K, 1), lambda: (0, 0)),
        ],
        out_shape=jax.ShapeDtypeStruct((Q, 8), jnp.int32),
    )(U, A, bcol)

    out = pl.pallas_call(
        _fused_body,
        grid=grid,
        in_specs=[
            pl.BlockSpec((Q, D), lambda i: (0, 0)),
            pl.BlockSpec((Q, D), lambda i: (0, 0)),
            pl.BlockSpec((Q, 8), lambda i: (0, 0)),
            pl.BlockSpec((_TN, D), lambda i: (i, 0)),
            pl.BlockSpec((D, _L * _K), lambda i: (0, 0)),
            pl.BlockSpec((_L * _K, 1), lambda i: (0, 0)),
            pl.BlockSpec((_L * _K, 1), lambda i: (0, 0)),
        ],
        out_shape=jax.ShapeDtypeStruct((Q, D), jnp.float32),
        scratch_shapes=[
            pltpu.VMEM((Q, D), jnp.float32),
            pltpu.VMEM((Q, 1), jnp.float32),
        ],
    )(U, Uc, qb, ref, A, bcol, mult)
    return out
